# baseline (speedup 1.0000x reference)
import functools

import jax
import jax.numpy as jnp
from jax import lax
from jax.experimental import pallas as pl
from jax.experimental.pallas import tpu as pltpu
from jax.experimental.pallas import tpu_sc as plsc

V = 1000000
D = 32
B = 16384
L = 50

NW = 32
BAGS_W = B // NW
CHUNK = 32
N_CHUNKS = BAGS_W // CHUNK
ROWS_C = CHUNK * L


def _fire_gathers(table_hbm, idx_v, vals_v, sem):
    handles = []
    for j in range(CHUNK):
        handles.append(pltpu.async_copy(
            table_hbm.at[idx_v.at[pl.ds(j * 128, L)]],
            vals_v.at[pl.ds(j * L, L)], sem))
    return handles


def _sc_bag_body(table_hbm, x_hbm, emb_hbm,
                 idx0, idx1, vals0, vals1, emb0, emb1,
                 sem0, sem1, semo):
    wid = lax.axis_index("s") * 2 + lax.axis_index("c")
    nidx = BAGS_W * L
    idx_bufs = (idx0, idx1)
    vals_bufs = (vals0, vals1)
    emb_bufs = (emb0, emb1)
    sems = (sem0, sem1)

    def idx_src(c):
        return x_hbm.at[pl.ds((wid * BAGS_W + c * CHUNK) * 128, CHUNK * 128)]

    pltpu.sync_copy(idx_src(0), idx_bufs[0])
    handles = _fire_gathers(table_hbm, idx_bufs[0], vals_bufs[0], sems[0])
    pltpu.sync_copy(idx_src(1), idx_bufs[1])

    for c in range(N_CHUNKS):
        cur = c % 2
        nxt = (c + 1) % 2
        for h in handles:
            h.wait()
        if c + 1 < N_CHUNKS:
            handles = _fire_gathers(
                table_hbm, idx_bufs[nxt], vals_bufs[nxt], sems[nxt])
        vals_v = vals_bufs[cur]
        emb_v = emb_bufs[cur]
        if c >= 2:
            pltpu.make_async_copy(
                emb_v,
                emb_hbm.at[pl.ds((wid * BAGS_W + (c - 2) * CHUNK) * D,
                                 CHUNK * D)],
                semo).wait()

        def do_bag(j, _):
            r0 = j * L
            acc0 = vals_v[r0, pl.ds(0, 16)]
            acc1 = vals_v[r0, pl.ds(16, 16)]
            for r in range(1, L):
                acc0 = acc0 + vals_v[r0 + r, pl.ds(0, 16)]
                acc1 = acc1 + vals_v[r0 + r, pl.ds(16, 16)]
            emb_v[pl.ds(j * D, 16)] = acc0
            emb_v[pl.ds(j * D + 16, 16)] = acc1
            return 0

        lax.fori_loop(0, CHUNK, do_bag, 0)
        pltpu.async_copy(
            emb_v,
            emb_hbm.at[pl.ds((wid * BAGS_W + c * CHUNK) * D, CHUNK * D)],
            semo)
        if c + 2 < N_CHUNKS:
            pltpu.sync_copy(idx_src(c + 2), idx_bufs[cur])

    for c in (N_CHUNKS - 2, N_CHUNKS - 1):
        pltpu.make_async_copy(
            emb_bufs[c % 2],
            emb_hbm.at[pl.ds((wid * BAGS_W + c * CHUNK) * D, CHUNK * D)],
            semo).wait()


@functools.lru_cache(maxsize=1)
def _sc_bag():
    return functools.partial(
        pl.kernel,
        out_type=jax.ShapeDtypeStruct((B * D,), jnp.float32),
        mesh=plsc.VectorSubcoreMesh(core_axis_name="c", subcore_axis_name="s"),
        compiler_params=pltpu.CompilerParams(use_tc_tiling_on_sc=False),
        scratch_types=[
            pltpu.VMEM((CHUNK * 128,), jnp.int32),
            pltpu.VMEM((CHUNK * 128,), jnp.int32),
            pltpu.VMEM((ROWS_C, D), jnp.float32),
            pltpu.VMEM((ROWS_C, D), jnp.float32),
            pltpu.VMEM((CHUNK * D,), jnp.float32),
            pltpu.VMEM((CHUNK * D,), jnp.float32),
            pltpu.SemaphoreType.DMA,
            pltpu.SemaphoreType.DMA,
            pltpu.SemaphoreType.DMA,
        ],
    )(_sc_bag_body)


LIN_BAGS = 2048
LIN_FLAT = LIN_BAGS * D
LIN_GRID = B // LIN_BAGS


def _tc_lin_body(e_ref, w_ref, o_ref):
    e = e_ref[...].reshape(LIN_FLAT // 128, 128)
    o_ref[0] = jnp.dot(e, w_ref[...], preferred_element_type=jnp.float32)


_tc_lin = pl.pallas_call(
    _tc_lin_body,
    grid=(LIN_GRID,),
    in_specs=[
        pl.BlockSpec((LIN_FLAT,), lambda i: (i,)),
        pl.BlockSpec((128, 4), lambda i: (0, 0)),
    ],
    out_specs=pl.BlockSpec((1, LIN_FLAT // 128, 4), lambda i: (i, 0, 0)),
    out_shape=jax.ShapeDtypeStruct((LIN_GRID, LIN_FLAT // 128, 4), jnp.float32),
)


@jax.jit
def kernel(x, table, W, b):
    xp = jnp.pad(x, ((0, 0), (0, 128 - L))).reshape(B * 128)
    emb = _sc_bag()(table, xp)
    w0 = W[0]
    wmat = (jnp.eye(4, dtype=jnp.float32)[:, None, :] * w0[:, None]).reshape(128, 4)
    out = _tc_lin(emb, wmat).reshape(B)
    return out + b[0]


# speedup vs baseline: 1.0000x; 1.0000x over previous
import functools

import jax
import jax.numpy as jnp
from jax import lax
from jax.experimental import pallas as pl
from jax.experimental.pallas import tpu as pltpu
from jax.experimental.pallas import tpu_sc as plsc

V = 1000000
D = 32
B = 16384
L = 50

NW = 32
BAGS_W = B // NW
CHUNK = 32
N_CHUNKS = BAGS_W // CHUNK
ROWS_C = CHUNK * L


def _fire_gathers(table_hbm, idx_v, vals_v, sem):
    handles = []
    for j in range(CHUNK):
        handles.append(pltpu.async_copy(
            table_hbm.at[idx_v.at[pl.ds(j * 128, L)]],
            vals_v.at[pl.ds(j * L, L)], sem))
    return handles


def _sc_bag_body(table_hbm, x_hbm, emb_hbm,
                 idx0, idx1, vals0, vals1, emb0, emb1,
                 sem0, sem1, semo):
    wid = lax.axis_index("s") * 2 + lax.axis_index("c")
    nidx = BAGS_W * L
    idx_bufs = (idx0, idx1)
    vals_bufs = (vals0, vals1)
    emb_bufs = (emb0, emb1)
    sems = (sem0, sem1)

    def idx_src(c):
        return x_hbm.at[pl.ds((wid * BAGS_W + c * CHUNK) * 128, CHUNK * 128)]

    pltpu.sync_copy(idx_src(0), idx_bufs[0])
    handles = _fire_gathers(table_hbm, idx_bufs[0], vals_bufs[0], sems[0])
    pltpu.sync_copy(idx_src(1), idx_bufs[1])

    for c in range(N_CHUNKS):
        cur = c % 2
        nxt = (c + 1) % 2
        for h in handles:
            h.wait()
        if c + 1 < N_CHUNKS:
            handles = _fire_gathers(
                table_hbm, idx_bufs[nxt], vals_bufs[nxt], sems[nxt])
        vals_v = vals_bufs[cur]
        emb_v = emb_bufs[cur]
        if c >= 2:
            pltpu.make_async_copy(
                emb_v,
                emb_hbm.at[pl.ds((wid * BAGS_W + (c - 2) * CHUNK) * D,
                                 CHUNK * D)],
                semo).wait()

        def do_bag(j, _):
            r0 = j * L
            acc0 = vals_v[r0, pl.ds(0, 16)]
            acc1 = vals_v[r0, pl.ds(16, 16)]
            for r in range(1, L):
                acc0 = acc0 + vals_v[r0 + r, pl.ds(0, 16)]
                acc1 = acc1 + vals_v[r0 + r, pl.ds(16, 16)]
            emb_v[pl.ds(j * D, 16)] = acc0
            emb_v[pl.ds(j * D + 16, 16)] = acc1
            return 0

        lax.fori_loop(0, CHUNK, do_bag, 0)
        pltpu.async_copy(
            emb_v,
            emb_hbm.at[pl.ds((wid * BAGS_W + c * CHUNK) * D, CHUNK * D)],
            semo)
        if c + 2 < N_CHUNKS:
            pltpu.sync_copy(idx_src(c + 2), idx_bufs[cur])

    for c in (N_CHUNKS - 2, N_CHUNKS - 1):
        pltpu.make_async_copy(
            emb_bufs[c % 2],
            emb_hbm.at[pl.ds((wid * BAGS_W + c * CHUNK) * D, CHUNK * D)],
            semo).wait()


@functools.lru_cache(maxsize=1)
def _sc_bag():
    return functools.partial(
        pl.kernel,
        out_type=jax.ShapeDtypeStruct((B * D,), jnp.float32),
        mesh=plsc.VectorSubcoreMesh(core_axis_name="c", subcore_axis_name="s"),
        compiler_params=pltpu.CompilerParams(use_tc_tiling_on_sc=False),
        scratch_types=[
            pltpu.VMEM((CHUNK * 128,), jnp.int32),
            pltpu.VMEM((CHUNK * 128,), jnp.int32),
            pltpu.VMEM((ROWS_C, D), jnp.float32),
            pltpu.VMEM((ROWS_C, D), jnp.float32),
            pltpu.VMEM((CHUNK * D,), jnp.float32),
            pltpu.VMEM((CHUNK * D,), jnp.float32),
            pltpu.SemaphoreType.DMA,
            pltpu.SemaphoreType.DMA,
            pltpu.SemaphoreType.DMA,
        ],
    )(_sc_bag_body)


LIN_BAGS = 2048
LIN_FLAT = LIN_BAGS * D
LIN_GRID = B // LIN_BAGS


def _tc_lin_body(e_ref, w_ref, o_ref):
    e = e_ref[...].reshape(LIN_FLAT // 128, 128)
    o_ref[0] = jnp.dot(e, w_ref[...], preferred_element_type=jnp.float32)


_tc_lin = pl.pallas_call(
    _tc_lin_body,
    grid=(LIN_GRID,),
    in_specs=[
        pl.BlockSpec((LIN_FLAT,), lambda i: (i,)),
        pl.BlockSpec((128, 4), lambda i: (0, 0)),
    ],
    out_specs=pl.BlockSpec((1, LIN_FLAT // 128, 4), lambda i: (i, 0, 0)),
    out_shape=jax.ShapeDtypeStruct((LIN_GRID, LIN_FLAT // 128, 4), jnp.float32),
)


XF_ROWS = 2048
XF_GRID = B // XF_ROWS


def _tc_flat_body(x_ref, o_ref):
    xw = jnp.concatenate(
        [x_ref[...], jnp.zeros((XF_ROWS, 128 - L), jnp.int32)], axis=1)
    o_ref[...] = xw.reshape(XF_ROWS * 128)


_tc_flat = pl.pallas_call(
    _tc_flat_body,
    grid=(XF_GRID,),
    in_specs=[pl.BlockSpec((XF_ROWS, L), lambda i: (i, 0))],
    out_specs=pl.BlockSpec((XF_ROWS * 128,), lambda i: (i,)),
    out_shape=jax.ShapeDtypeStruct((B * 128,), jnp.int32),
)


@jax.jit
def kernel(x, table, W, b):
    xf = _tc_flat(x)
    emb = _sc_bag()(table, xf)
    w0 = W[0]
    wmat = (jnp.eye(4, dtype=jnp.float32)[:, None, :] * w0[:, None]).reshape(128, 4)
    out = _tc_lin(emb, wmat).reshape(B)
    return out + b[0]


# speedup vs baseline: 1.0101x; 1.0100x over previous
import functools

import jax
import jax.numpy as jnp
from jax import lax
from jax.experimental import pallas as pl
from jax.experimental.pallas import tpu as pltpu
from jax.experimental.pallas import tpu_sc as plsc

V = 1000000
D = 32
B = 16384
L = 50

NW = 32
BAGS_W = B // NW
CHUNK = 32
N_CHUNKS = BAGS_W // CHUNK
ROWS_C = CHUNK * L


FULL_STREAMS = ROWS_C // 128
TAIL = ROWS_C - FULL_STREAMS * 128


def _fire_gathers(table_hbm, idx_v, vals_v, sem):
    handles = []
    for s in range(FULL_STREAMS):
        handles.append(pltpu.async_copy(
            table_hbm.at[idx_v.at[pl.ds(s * 128, 128)]],
            vals_v.at[pl.ds(s * 128, 128)], sem))
    handles.append(pltpu.async_copy(
        table_hbm.at[idx_v.at[pl.ds(FULL_STREAMS * 128, TAIL)]],
        vals_v.at[pl.ds(FULL_STREAMS * 128, TAIL)], sem))
    return handles


def _sc_bag_body(table_hbm, x_hbm, emb_hbm,
                 idx0, idx1, vals0, vals1, emb0, emb1,
                 sem0, sem1, semo):
    wid = lax.axis_index("s") * 2 + lax.axis_index("c")
    nidx = BAGS_W * L
    idx_bufs = (idx0, idx1)
    vals_bufs = (vals0, vals1)
    emb_bufs = (emb0, emb1)
    sems = (sem0, sem1)

    def idx_src(c):
        return x_hbm.at[pl.ds(wid * nidx + c * ROWS_C, ROWS_C)]

    pltpu.sync_copy(idx_src(0), idx_bufs[0])
    handles = _fire_gathers(table_hbm, idx_bufs[0], vals_bufs[0], sems[0])
    pltpu.sync_copy(idx_src(1), idx_bufs[1])

    for c in range(N_CHUNKS):
        cur = c % 2
        nxt = (c + 1) % 2
        for h in handles:
            h.wait()
        if c + 1 < N_CHUNKS:
            handles = _fire_gathers(
                table_hbm, idx_bufs[nxt], vals_bufs[nxt], sems[nxt])
        vals_v = vals_bufs[cur]
        emb_v = emb_bufs[cur]
        if c >= 2:
            pltpu.make_async_copy(
                emb_v,
                emb_hbm.at[pl.ds((wid * BAGS_W + (c - 2) * CHUNK) * D,
                                 CHUNK * D)],
                semo).wait()

        def do_bag(j, _):
            r0 = j * L
            acc0 = vals_v[r0, pl.ds(0, 16)]
            acc1 = vals_v[r0, pl.ds(16, 16)]
            for r in range(1, L):
                acc0 = acc0 + vals_v[r0 + r, pl.ds(0, 16)]
                acc1 = acc1 + vals_v[r0 + r, pl.ds(16, 16)]
            emb_v[pl.ds(j * D, 16)] = acc0
            emb_v[pl.ds(j * D + 16, 16)] = acc1
            return 0

        lax.fori_loop(0, CHUNK, do_bag, 0)
        pltpu.async_copy(
            emb_v,
            emb_hbm.at[pl.ds((wid * BAGS_W + c * CHUNK) * D, CHUNK * D)],
            semo)
        if c + 2 < N_CHUNKS:
            pltpu.sync_copy(idx_src(c + 2), idx_bufs[cur])

    for c in (N_CHUNKS - 2, N_CHUNKS - 1):
        pltpu.make_async_copy(
            emb_bufs[c % 2],
            emb_hbm.at[pl.ds((wid * BAGS_W + c * CHUNK) * D, CHUNK * D)],
            semo).wait()


@functools.lru_cache(maxsize=1)
def _sc_bag():
    return functools.partial(
        pl.kernel,
        out_type=jax.ShapeDtypeStruct((B * D,), jnp.float32),
        mesh=plsc.VectorSubcoreMesh(core_axis_name="c", subcore_axis_name="s"),
        compiler_params=pltpu.CompilerParams(use_tc_tiling_on_sc=False),
        scratch_types=[
            pltpu.VMEM((ROWS_C,), jnp.int32),
            pltpu.VMEM((ROWS_C,), jnp.int32),
            pltpu.VMEM((ROWS_C, D), jnp.float32),
            pltpu.VMEM((ROWS_C, D), jnp.float32),
            pltpu.VMEM((CHUNK * D,), jnp.float32),
            pltpu.VMEM((CHUNK * D,), jnp.float32),
            pltpu.SemaphoreType.DMA,
            pltpu.SemaphoreType.DMA,
            pltpu.SemaphoreType.DMA,
        ],
    )(_sc_bag_body)


LIN_BAGS = 2048
LIN_FLAT = LIN_BAGS * D
LIN_GRID = B // LIN_BAGS


def _tc_lin_body(b_ref, e_ref, w_ref, o_ref):
    e = e_ref[...].reshape(LIN_FLAT // 128, 128)
    o_ref[0] = jnp.dot(e, w_ref[...],
                       preferred_element_type=jnp.float32) + b_ref[0]


_tc_lin = pl.pallas_call(
    _tc_lin_body,
    grid=(LIN_GRID,),
    in_specs=[
        pl.BlockSpec(memory_space=pltpu.SMEM),
        pl.BlockSpec((LIN_FLAT,), lambda i: (i,)),
        pl.BlockSpec((128, 4), lambda i: (0, 0)),
    ],
    out_specs=pl.BlockSpec((1, LIN_FLAT // 128, 4), lambda i: (i, 0, 0)),
    out_shape=jax.ShapeDtypeStruct((LIN_GRID, LIN_FLAT // 128, 4), jnp.float32),
)


@jax.jit
def kernel(x, table, W, b):
    xf = x.reshape(B * L)
    emb = _sc_bag()(table, xf)
    w0 = W[0]
    wmat = (jnp.eye(4, dtype=jnp.float32)[:, None, :] * w0[:, None]).reshape(128, 4)
    return _tc_lin(b, emb, wmat).reshape(B)
